# Initial kernel scaffold; baseline (speedup 1.0000x reference)
import functools

import jax
import jax.numpy as jnp
from jax import lax
from jax.experimental import pallas as pl
from jax.experimental.pallas import tpu as pltpu
from jax.experimental.pallas import tpu_sc as plsc

N = 10000
N_PAD = 10240
HD = 128
ZD = 64
NC, NS = 2, 16
NW = NC * NS
CB = 128
RP = N_PAD // NS
BR = 1024
GRID = N_PAD // BR

_MESH = plsc.VectorSubcoreMesh(core_axis_name="c", subcore_axis_name="s")


def _make_spmv(nchunk):
    @functools.partial(
        pl.kernel, mesh=_MESH,
        out_type=jax.ShapeDtypeStruct((NC * N_PAD, HD), jnp.float32),
        scratch_types=[
            pltpu.VMEM((nchunk, CB), jnp.int32),
            pltpu.VMEM((nchunk, CB), jnp.int32),
            pltpu.VMEM((CB, HD), jnp.float32),
            pltpu.VMEM_SHARED((N_PAD, HD), jnp.float32),
            pltpu.SemaphoreType.DMA,
        ])
    def spmv(x_hbm, src_hbm, dst_hbm, zeros_hbm, out_hbm,
             src_v, dst_v, rows_v, acc, sem):
        c = lax.axis_index("c")
        s = lax.axis_index("s")
        w = s * NC + c
        pltpu.sync_copy(zeros_hbm, acc.at[pl.ds(s * RP, RP)])
        pltpu.sync_copy(src_hbm.at[w], src_v)
        pltpu.sync_copy(dst_hbm.at[w], dst_v)
        plsc.subcore_barrier()

        def body(j, carry):
            pltpu.async_copy(x_hbm.at[src_v.at[j]], rows_v, sem).wait()
            pltpu.sync_copy(rows_v, acc.at[dst_v.at[j]], add=True)
            return carry

        lax.fori_loop(0, nchunk, body, 0)
        plsc.subcore_barrier()
        pltpu.sync_copy(acc.at[pl.ds(s * RP, RP)],
                        out_hbm.at[pl.ds(c * N_PAD + s * RP, RP)])

    return spmv


def _make_deg(nchunk):
    @functools.partial(
        pl.kernel, mesh=_MESH,
        out_type=jax.ShapeDtypeStruct((NC * N_PAD, 16), jnp.float32),
        scratch_types=[
            pltpu.VMEM((nchunk, CB), jnp.int32),
            pltpu.VMEM((CB, 16), jnp.float32),
            pltpu.VMEM_SHARED((N_PAD, 16), jnp.float32),
        ])
    def deg(dst_hbm, ones_hbm, zeros_hbm, out_hbm, dst_v, ones_v, acc):
        c = lax.axis_index("c")
        s = lax.axis_index("s")
        w = s * NC + c
        pltpu.sync_copy(zeros_hbm, acc.at[pl.ds(s * RP, RP)])
        pltpu.sync_copy(dst_hbm.at[w], dst_v)
        pltpu.sync_copy(ones_hbm, ones_v)
        plsc.subcore_barrier()

        def body(j, carry):
            pltpu.sync_copy(ones_v, acc.at[dst_v.at[j]], add=True)
            return carry

        lax.fori_loop(0, nchunk, body, 0)
        plsc.subcore_barrier()
        pltpu.sync_copy(acc.at[pl.ds(s * RP, RP)],
                        out_hbm.at[pl.ds(c * N_PAD + s * RP, RP)])

    return deg


def _softplus(v):
    return jnp.log1p(jnp.exp(-jnp.abs(v))) + jnp.maximum(v, 0.0)


def _row(cols):
    return pl.BlockSpec((BR, cols), lambda i: (i, 0))


def _full(shape):
    return pl.BlockSpec(shape, lambda i: tuple(0 for _ in shape))


def _part_spec(cols):
    return pl.BlockSpec((NC, BR, cols), lambda i: (0, i, 0))


def _mm(a, b):
    return jnp.dot(a, b, preferred_element_type=jnp.float32)


def _d1_body(x_ref, h0_ref, wpx, bpx, wpr, bpr, wpm, bpm, wps, bps,
             phi_x_ref, pm_ref, ps_ref):
    x = x_ref[...]
    h0 = h0_ref[...]
    phi_x_ref[...] = jnp.maximum(_mm(x, wpx[...]) + bpx[...], 0.0)
    pt = jnp.maximum(_mm(h0, wpr[...]) + bpr[...], 0.0)
    pm_ref[...] = _mm(pt, wpm[...]) + bpm[...]
    ps_ref[...] = _softplus(_mm(pt, wps[...]) + bps[...])


def _d2_body(degp_ref, phi_x_ref, h0_ref, wea, web, dinv_ref, u1_ref):
    deg = degp_ref[0, :, 0:1] + degp_ref[1, :, 0:1] + 1.0
    dinv = 1.0 / jnp.sqrt(jnp.clip(deg, 1.0))
    dinv_ref[...] = jnp.broadcast_to(dinv, dinv_ref.shape)
    u1_ref[...] = dinv * (_mm(phi_x_ref[...], wea[...]) +
                          _mm(h0_ref[...], web[...]))


def _d3_body(s1_ref, u1_ref, dinv_ref, wem, wes, benc, u2_ref):
    dinv = dinv_ref[:, 0:1]
    enc = dinv * (s1_ref[0] + s1_ref[1] + u1_ref[...]) + benc[...]
    u2_ref[...] = dinv * jnp.concatenate(
        [_mm(enc, wem[...]), _mm(enc, wes[...])], axis=1)


def _d4_body(s2_ref, u2_ref, dinv_ref, noise_ref, bem, bes, phi_x_ref,
             h0_ref, wphz, bphz, wxza, wxzb, whz, wxra, wxrb, whr,
             wxha, wxhb, em_ref, es_ref, u3a_ref, u3b_ref, u3c_ref):
    dinv = dinv_ref[:, 0:1]
    t = dinv * (s2_ref[0] + s2_ref[1] + u2_ref[...])
    em = t[:, :ZD] + bem[...]
    es = _softplus(t[:, ZD:] + bes[...])
    em_ref[...] = em
    es_ref[...] = es
    z = em + es * noise_ref[...]
    phi_z = jnp.maximum(_mm(z, wphz[...]) + bphz[...], 0.0)
    phi_x = phi_x_ref[...]
    h0 = h0_ref[...]
    u3a_ref[...] = dinv * (_mm(phi_x, wxza[...]) + _mm(phi_z, wxzb[...]) +
                           _mm(h0, whz[...]))
    u3b_ref[...] = dinv * (_mm(phi_x, wxra[...]) + _mm(phi_z, wxrb[...]) +
                           _mm(h0, whr[...]))
    u3c_ref[...] = dinv * (_mm(phi_x, wxha[...]) + _mm(phi_z, wxhb[...]))


def _d5_body(s3a_ref, s3b_ref, u3a_ref, u3b_ref, dinv_ref, h0_ref, whh,
             zg_ref, u4_ref):
    dinv = dinv_ref[:, 0:1]
    zg = jax.nn.sigmoid(dinv * (s3a_ref[0] + s3a_ref[1] + u3a_ref[...]))
    rg = jax.nn.sigmoid(dinv * (s3b_ref[0] + s3b_ref[1] + u3b_ref[...]))
    zg_ref[...] = zg
    u4_ref[...] = dinv * _mm(rg * h0_ref[...], whh[...])


def _d6_body(s3c_ref, u3c_ref, s4_ref, u4_ref, dinv_ref, zg_ref, h0_ref,
             hout_ref):
    dinv = dinv_ref[:, 0:1]
    ht = jnp.tanh(dinv * (s3c_ref[0] + s3c_ref[1] + u3c_ref[...]) +
                  dinv * (s4_ref[0] + s4_ref[1] + u4_ref[...]))
    zg = zg_ref[...]
    hout_ref[...] = zg * h0_ref[...] + (1.0 - zg) * ht


def _call(body, in_specs, out_specs, out_shapes, args):
    return pl.pallas_call(
        body, grid=(GRID,), in_specs=in_specs, out_specs=out_specs,
        out_shape=out_shapes)(*args)


def kernel(x, edge_index, h, noise, W_phi_x, b_phi_x, W_enc, b_enc,
           W_enc_mean, b_enc_mean, W_enc_std, b_enc_std,
           W_prior, b_prior, W_prior_mean, b_prior_mean,
           W_prior_std, b_prior_std, W_phi_z, b_phi_z,
           W_xz, W_hz, W_xr, W_hr, W_xh, W_hh):
    E = edge_index.shape[1]
    EC = NW * CB
    nchunk = -(-E // EC)
    e_pad = nchunk * EC

    f32 = jnp.float32
    padn = N_PAD - N
    xp = jnp.pad(x, ((0, padn), (0, 0)))
    h0p = jnp.pad(h[0], ((0, padn), (0, 0)))
    noisep = jnp.pad(noise, ((0, padn), (0, 0)))

    pad_idx = N + (jnp.arange(e_pad - E, dtype=jnp.int32) % padn)
    srcp = jnp.concatenate([edge_index[0], pad_idx]).reshape(NW, nchunk, CB)
    dstp = jnp.concatenate([edge_index[1], pad_idx]).reshape(NW, nchunk, CB)

    zeros128 = jnp.zeros((RP, HD), f32)
    zeros16 = jnp.zeros((RP, 16), f32)
    ones16 = jnp.ones((CB, 16), f32)

    spmv = _make_spmv(nchunk)
    degk = _make_deg(nchunk)

    def b2(v):
        return v.reshape(1, -1)

    degp = degk(dstp, ones16, zeros16).reshape(NC, N_PAD, 16)

    phi_x, pm, ps = _call(
        _d1_body,
        [_row(HD), _row(HD), _full((HD, HD)), _full((1, HD)),
         _full((HD, HD)), _full((1, HD)), _full((HD, ZD)), _full((1, ZD)),
         _full((HD, ZD)), _full((1, ZD))],
        [_row(HD), _row(ZD), _row(ZD)],
        [jax.ShapeDtypeStruct((N_PAD, HD), f32),
         jax.ShapeDtypeStruct((N_PAD, ZD), f32),
         jax.ShapeDtypeStruct((N_PAD, ZD), f32)],
        (xp, h0p, W_phi_x, b2(b_phi_x), W_prior, b2(b_prior),
         W_prior_mean, b2(b_prior_mean), W_prior_std, b2(b_prior_std)))

    dinv16, U1 = _call(
        _d2_body,
        [_part_spec(16), _row(HD), _row(HD), _full((HD, HD)),
         _full((HD, HD))],
        [_row(16), _row(HD)],
        [jax.ShapeDtypeStruct((N_PAD, 16), f32),
         jax.ShapeDtypeStruct((N_PAD, HD), f32)],
        (degp, phi_x, h0p, W_enc[:HD], W_enc[HD:]))

    S1 = spmv(U1, srcp, dstp, zeros128).reshape(NC, N_PAD, HD)
    U2 = _call(
        _d3_body,
        [_part_spec(HD), _row(HD), _row(16), _full((HD, ZD)),
         _full((HD, ZD)), _full((1, HD))],
        _row(HD),
        jax.ShapeDtypeStruct((N_PAD, HD), f32),
        (S1, U1, dinv16, W_enc_mean, W_enc_std, b2(b_enc)))

    S2 = spmv(U2, srcp, dstp, zeros128).reshape(NC, N_PAD, HD)
    em, es, U3a, U3b, U3c = _call(
        _d4_body,
        [_part_spec(HD), _row(HD), _row(16), _row(ZD), _full((1, ZD)),
         _full((1, ZD)), _row(HD), _row(HD), _full((ZD, HD)),
         _full((1, HD)), _full((HD, HD)), _full((HD, HD)), _full((HD, HD)),
         _full((HD, HD)), _full((HD, HD)), _full((HD, HD)),
         _full((HD, HD)), _full((HD, HD))],
        [_row(ZD), _row(ZD), _row(HD), _row(HD), _row(HD)],
        [jax.ShapeDtypeStruct((N_PAD, ZD), f32),
         jax.ShapeDtypeStruct((N_PAD, ZD), f32),
         jax.ShapeDtypeStruct((N_PAD, HD), f32),
         jax.ShapeDtypeStruct((N_PAD, HD), f32),
         jax.ShapeDtypeStruct((N_PAD, HD), f32)],
        (S2, U2, dinv16, noisep, b2(b_enc_mean), b2(b_enc_std), phi_x, h0p,
         W_phi_z, b2(b_phi_z), W_xz[:HD], W_xz[HD:], W_hz,
         W_xr[:HD], W_xr[HD:], W_hr, W_xh[:HD], W_xh[HD:]))

    S3a = spmv(U3a, srcp, dstp, zeros128).reshape(NC, N_PAD, HD)
    S3b = spmv(U3b, srcp, dstp, zeros128).reshape(NC, N_PAD, HD)
    S3c = spmv(U3c, srcp, dstp, zeros128).reshape(NC, N_PAD, HD)
    zg, U4 = _call(
        _d5_body,
        [_part_spec(HD), _part_spec(HD), _row(HD), _row(HD), _row(16),
         _row(HD), _full((HD, HD))],
        [_row(HD), _row(HD)],
        [jax.ShapeDtypeStruct((N_PAD, HD), f32),
         jax.ShapeDtypeStruct((N_PAD, HD), f32)],
        (S3a, S3b, U3a, U3b, dinv16, h0p, W_hh))

    S4 = spmv(U4, srcp, dstp, zeros128).reshape(NC, N_PAD, HD)
    hout = _call(
        _d6_body,
        [_part_spec(HD), _row(HD), _part_spec(HD), _row(HD), _row(16),
         _row(HD), _row(HD)],
        _row(HD),
        jax.ShapeDtypeStruct((N_PAD, HD), f32),
        (S3c, U3c, S4, U4, dinv16, zg, h0p))

    return (em[:N], es[:N], pm[:N], ps[:N], hout[:N][None])


# speedup vs baseline: 18.2494x; 18.2494x over previous
import functools

import jax
import jax.numpy as jnp
from jax import lax
from jax.experimental import pallas as pl
from jax.experimental.pallas import tpu as pltpu
from jax.experimental.pallas import tpu_sc as plsc

N = 10000
N_PAD = 10240
HD = 128
ZD = 64
NC, NS = 2, 16
NW = NC * NS
CB = 128
RP = N_PAD // NS
BR = 1024
GRID = N_PAD // BR

_MESH = plsc.VectorSubcoreMesh(core_axis_name="c", subcore_axis_name="s")


def _make_spmv(nchunk):
    @functools.partial(
        pl.kernel, mesh=_MESH,
        out_type=jax.ShapeDtypeStruct((NC * N_PAD, HD), jnp.float32),
        scratch_types=[
            pltpu.VMEM((nchunk, CB), jnp.int32),
            pltpu.VMEM((nchunk, CB), jnp.int32),
            pltpu.VMEM((CB, HD), jnp.float32),
            pltpu.VMEM_SHARED((N_PAD, HD), jnp.float32),
            pltpu.SemaphoreType.DMA,
        ])
    def spmv(x_hbm, src_hbm, dst_hbm, zeros_hbm, out_hbm,
             src_v, dst_v, rows_v, acc, sem):
        c = lax.axis_index("c")
        s = lax.axis_index("s")
        w = s * NC + c
        pltpu.sync_copy(zeros_hbm, acc.at[pl.ds(s * RP, RP)])
        pltpu.sync_copy(src_hbm.at[w], src_v)
        pltpu.sync_copy(dst_hbm.at[w], dst_v)
        plsc.subcore_barrier()

        def body(j, carry):
            pltpu.async_copy(x_hbm.at[src_v.at[j]], rows_v, sem).wait()
            pltpu.sync_copy(rows_v, acc.at[dst_v.at[j]], add=True)
            return carry

        lax.fori_loop(0, nchunk, body, 0)
        plsc.subcore_barrier()
        pltpu.sync_copy(acc.at[pl.ds(s * RP, RP)],
                        out_hbm.at[pl.ds(c * N_PAD + s * RP, RP)])

    return spmv


def _make_deg(nchunk):
    @functools.partial(
        pl.kernel, mesh=_MESH,
        out_type=jax.ShapeDtypeStruct((NC * N_PAD, HD), jnp.float32),
        scratch_types=[
            pltpu.VMEM((nchunk, CB), jnp.int32),
            pltpu.VMEM((CB, HD), jnp.float32),
            pltpu.VMEM_SHARED((N_PAD, HD), jnp.float32),
        ])
    def deg(dst_hbm, ones_hbm, zeros_hbm, out_hbm, dst_v, ones_v, acc):
        c = lax.axis_index("c")
        s = lax.axis_index("s")
        w = s * NC + c
        pltpu.sync_copy(zeros_hbm, acc.at[pl.ds(s * RP, RP)])
        pltpu.sync_copy(dst_hbm.at[w], dst_v)
        pltpu.sync_copy(ones_hbm, ones_v)
        plsc.subcore_barrier()

        def body(j, carry):
            pltpu.sync_copy(ones_v, acc.at[dst_v.at[j]], add=True)
            return carry

        lax.fori_loop(0, nchunk, body, 0)
        plsc.subcore_barrier()
        pltpu.sync_copy(acc.at[pl.ds(s * RP, RP)],
                        out_hbm.at[pl.ds(c * N_PAD + s * RP, RP)])

    return deg


def _softplus(v):
    return jnp.log1p(jnp.exp(-jnp.abs(v))) + jnp.maximum(v, 0.0)


def _row(cols):
    return pl.BlockSpec((BR, cols), lambda i: (i, 0))


def _full(shape):
    return pl.BlockSpec(shape, lambda i: tuple(0 for _ in shape))


def _part_spec(cols):
    return pl.BlockSpec((NC, BR, cols), lambda i: (0, i, 0))


def _mm(a, b):
    return jnp.dot(a, b, preferred_element_type=jnp.float32)


def _d1_body(x_ref, h0_ref, wpx, bpx, wpr, bpr, wpm, bpm, wps, bps,
             phi_x_ref, pm_ref, ps_ref):
    x = x_ref[...]
    h0 = h0_ref[...]
    phi_x_ref[...] = jnp.maximum(_mm(x, wpx[...]) + bpx[...], 0.0)
    pt = jnp.maximum(_mm(h0, wpr[...]) + bpr[...], 0.0)
    pm_ref[...] = _mm(pt, wpm[...]) + bpm[...]
    ps_ref[...] = _softplus(_mm(pt, wps[...]) + bps[...])


def _d2_body(degp_ref, phi_x_ref, h0_ref, wea, web, dinv_ref, u1_ref):
    deg = degp_ref[0, :, 0:1] + degp_ref[1, :, 0:1] + 1.0
    dinv = 1.0 / jnp.sqrt(jnp.clip(deg, 1.0))
    dinv_ref[...] = jnp.broadcast_to(dinv, dinv_ref.shape)
    u1_ref[...] = dinv * (_mm(phi_x_ref[...], wea[...]) +
                          _mm(h0_ref[...], web[...]))


def _d3_body(s1_ref, u1_ref, dinv_ref, wem, wes, benc, u2_ref):
    dinv = dinv_ref[:, 0:1]
    enc = dinv * (s1_ref[0] + s1_ref[1] + u1_ref[...]) + benc[...]
    u2_ref[...] = dinv * jnp.concatenate(
        [_mm(enc, wem[...]), _mm(enc, wes[...])], axis=1)


def _d4_body(s2_ref, u2_ref, dinv_ref, noise_ref, bem, bes, phi_x_ref,
             h0_ref, wphz, bphz, wxza, wxzb, whz, wxra, wxrb, whr,
             wxha, wxhb, em_ref, es_ref, u3a_ref, u3b_ref, u3c_ref):
    dinv = dinv_ref[:, 0:1]
    t = dinv * (s2_ref[0] + s2_ref[1] + u2_ref[...])
    em = t[:, :ZD] + bem[...]
    es = _softplus(t[:, ZD:] + bes[...])
    em_ref[...] = em
    es_ref[...] = es
    z = em + es * noise_ref[...]
    phi_z = jnp.maximum(_mm(z, wphz[...]) + bphz[...], 0.0)
    phi_x = phi_x_ref[...]
    h0 = h0_ref[...]
    u3a_ref[...] = dinv * (_mm(phi_x, wxza[...]) + _mm(phi_z, wxzb[...]) +
                           _mm(h0, whz[...]))
    u3b_ref[...] = dinv * (_mm(phi_x, wxra[...]) + _mm(phi_z, wxrb[...]) +
                           _mm(h0, whr[...]))
    u3c_ref[...] = dinv * (_mm(phi_x, wxha[...]) + _mm(phi_z, wxhb[...]))


def _d5_body(s3a_ref, s3b_ref, u3a_ref, u3b_ref, dinv_ref, h0_ref, whh,
             zg_ref, u4_ref):
    dinv = dinv_ref[:, 0:1]
    zg = jax.nn.sigmoid(dinv * (s3a_ref[0] + s3a_ref[1] + u3a_ref[...]))
    rg = jax.nn.sigmoid(dinv * (s3b_ref[0] + s3b_ref[1] + u3b_ref[...]))
    zg_ref[...] = zg
    u4_ref[...] = dinv * _mm(rg * h0_ref[...], whh[...])


def _d6_body(s3c_ref, u3c_ref, s4_ref, u4_ref, dinv_ref, zg_ref, h0_ref,
             hout_ref):
    dinv = dinv_ref[:, 0:1]
    ht = jnp.tanh(dinv * (s3c_ref[0] + s3c_ref[1] + u3c_ref[...]) +
                  dinv * (s4_ref[0] + s4_ref[1] + u4_ref[...]))
    zg = zg_ref[...]
    hout_ref[...] = zg * h0_ref[...] + (1.0 - zg) * ht


def _call(body, in_specs, out_specs, out_shapes, args):
    return pl.pallas_call(
        body, grid=(GRID,), in_specs=in_specs, out_specs=out_specs,
        out_shape=out_shapes)(*args)


def kernel(x, edge_index, h, noise, W_phi_x, b_phi_x, W_enc, b_enc,
           W_enc_mean, b_enc_mean, W_enc_std, b_enc_std,
           W_prior, b_prior, W_prior_mean, b_prior_mean,
           W_prior_std, b_prior_std, W_phi_z, b_phi_z,
           W_xz, W_hz, W_xr, W_hr, W_xh, W_hh):
    E = edge_index.shape[1]
    EC = NW * CB
    nchunk = -(-E // EC)
    e_pad = nchunk * EC

    f32 = jnp.float32
    padn = N_PAD - N
    xp = jnp.pad(x, ((0, padn), (0, 0)))
    h0p = jnp.pad(h[0], ((0, padn), (0, 0)))
    noisep = jnp.pad(noise, ((0, padn), (0, 0)))

    pad_idx = N + (jnp.arange(e_pad - E, dtype=jnp.int32) % padn)
    srcp = jnp.concatenate([edge_index[0], pad_idx]).reshape(NW, nchunk, CB)
    dstp = jnp.concatenate([edge_index[1], pad_idx]).reshape(NW, nchunk, CB)

    zeros128 = jnp.zeros((RP, HD), f32)
    ones128 = jnp.ones((CB, HD), f32)

    spmv = _make_spmv(nchunk)
    degk = _make_deg(nchunk)

    def b2(v):
        return v.reshape(1, -1)

    degp = degk(dstp, ones128, zeros128).reshape(NC, N_PAD, HD)

    phi_x, pm, ps = _call(
        _d1_body,
        [_row(HD), _row(HD), _full((HD, HD)), _full((1, HD)),
         _full((HD, HD)), _full((1, HD)), _full((HD, ZD)), _full((1, ZD)),
         _full((HD, ZD)), _full((1, ZD))],
        [_row(HD), _row(ZD), _row(ZD)],
        [jax.ShapeDtypeStruct((N_PAD, HD), f32),
         jax.ShapeDtypeStruct((N_PAD, ZD), f32),
         jax.ShapeDtypeStruct((N_PAD, ZD), f32)],
        (xp, h0p, W_phi_x, b2(b_phi_x), W_prior, b2(b_prior),
         W_prior_mean, b2(b_prior_mean), W_prior_std, b2(b_prior_std)))

    dinv16, U1 = _call(
        _d2_body,
        [_part_spec(HD), _row(HD), _row(HD), _full((HD, HD)),
         _full((HD, HD))],
        [_row(16), _row(HD)],
        [jax.ShapeDtypeStruct((N_PAD, 16), f32),
         jax.ShapeDtypeStruct((N_PAD, HD), f32)],
        (degp, phi_x, h0p, W_enc[:HD], W_enc[HD:]))

    S1 = spmv(U1, srcp, dstp, zeros128).reshape(NC, N_PAD, HD)
    U2 = _call(
        _d3_body,
        [_part_spec(HD), _row(HD), _row(16), _full((HD, ZD)),
         _full((HD, ZD)), _full((1, HD))],
        _row(HD),
        jax.ShapeDtypeStruct((N_PAD, HD), f32),
        (S1, U1, dinv16, W_enc_mean, W_enc_std, b2(b_enc)))

    S2 = spmv(U2, srcp, dstp, zeros128).reshape(NC, N_PAD, HD)
    em, es, U3a, U3b, U3c = _call(
        _d4_body,
        [_part_spec(HD), _row(HD), _row(16), _row(ZD), _full((1, ZD)),
         _full((1, ZD)), _row(HD), _row(HD), _full((ZD, HD)),
         _full((1, HD)), _full((HD, HD)), _full((HD, HD)), _full((HD, HD)),
         _full((HD, HD)), _full((HD, HD)), _full((HD, HD)),
         _full((HD, HD)), _full((HD, HD))],
        [_row(ZD), _row(ZD), _row(HD), _row(HD), _row(HD)],
        [jax.ShapeDtypeStruct((N_PAD, ZD), f32),
         jax.ShapeDtypeStruct((N_PAD, ZD), f32),
         jax.ShapeDtypeStruct((N_PAD, HD), f32),
         jax.ShapeDtypeStruct((N_PAD, HD), f32),
         jax.ShapeDtypeStruct((N_PAD, HD), f32)],
        (S2, U2, dinv16, noisep, b2(b_enc_mean), b2(b_enc_std), phi_x, h0p,
         W_phi_z, b2(b_phi_z), W_xz[:HD], W_xz[HD:], W_hz,
         W_xr[:HD], W_xr[HD:], W_hr, W_xh[:HD], W_xh[HD:]))

    S3a = spmv(U3a, srcp, dstp, zeros128).reshape(NC, N_PAD, HD)
    S3b = spmv(U3b, srcp, dstp, zeros128).reshape(NC, N_PAD, HD)
    S3c = spmv(U3c, srcp, dstp, zeros128).reshape(NC, N_PAD, HD)
    zg, U4 = _call(
        _d5_body,
        [_part_spec(HD), _part_spec(HD), _row(HD), _row(HD), _row(16),
         _row(HD), _full((HD, HD))],
        [_row(HD), _row(HD)],
        [jax.ShapeDtypeStruct((N_PAD, HD), f32),
         jax.ShapeDtypeStruct((N_PAD, HD), f32)],
        (S3a, S3b, U3a, U3b, dinv16, h0p, W_hh))

    S4 = spmv(U4, srcp, dstp, zeros128).reshape(NC, N_PAD, HD)
    hout = _call(
        _d6_body,
        [_part_spec(HD), _row(HD), _part_spec(HD), _row(HD), _row(16),
         _row(HD), _row(HD)],
        _row(HD),
        jax.ShapeDtypeStruct((N_PAD, HD), f32),
        (S3c, U3c, S4, U4, dinv16, zg, h0p))

    return (em[:N], es[:N], pm[:N], ps[:N], hout[:N][None])


# speedup vs baseline: 21.1012x; 1.1563x over previous
import functools

import jax
import jax.numpy as jnp
from jax import lax
from jax.experimental import pallas as pl
from jax.experimental.pallas import tpu as pltpu
from jax.experimental.pallas import tpu_sc as plsc

N = 10000
N_PAD = 10240
HD = 128
ZD = 64
NC, NS = 2, 16
NW = NC * NS
CB = 128
RP = N_PAD // NS
BR = 1024
GRID = N_PAD // BR

_MESH = plsc.VectorSubcoreMesh(core_axis_name="c", subcore_axis_name="s")


def _make_spmv(nchunk):
    assert nchunk % 4 == 0
    half = nchunk // 2

    @functools.partial(
        pl.kernel, mesh=_MESH,
        out_type=jax.ShapeDtypeStruct((NC * N_PAD, HD), jnp.float32),
        scratch_types=[
            pltpu.VMEM((half, CB), jnp.int32),
            pltpu.VMEM((half, CB), jnp.int32),
            pltpu.VMEM((CB, HD), jnp.float32),
            pltpu.VMEM((CB, HD), jnp.float32),
            pltpu.SemaphoreType.DMA,
            pltpu.SemaphoreType.DMA,
            pltpu.SemaphoreType.DMA,
            pltpu.SemaphoreType.DMA,
            pltpu.VMEM_SHARED((N_PAD, HD), jnp.float32),
        ])
    def spmv(x_hbm, src_hbm, dst_hbm, zeros_hbm, out_hbm,
             src_v, dst_v, b0, b1, g0, g1, s0, s1, acc):
        c = lax.axis_index("c")
        s = lax.axis_index("s")
        w = s * NC + c
        pltpu.sync_copy(zeros_hbm, acc.at[pl.ds(s * RP, RP)])
        plsc.subcore_barrier()
        bufs = (b0, b1)
        gsems = (g0, g1)
        ssems = (s0, s1)

        for phase in range(2):
            base = w * nchunk + phase * half
            pltpu.sync_copy(src_hbm.at[pl.ds(base, half)], src_v)
            pltpu.sync_copy(dst_hbm.at[pl.ds(base, half)], dst_v)
            pltpu.async_copy(x_hbm.at[src_v.at[0]], b0, g0)
            pltpu.async_copy(x_hbm.at[src_v.at[1]], b1, g1)

            def body(j2, carry):
                j = j2 * 2
                for k in range(2):
                    pltpu.make_async_copy(x_hbm.at[src_v.at[j + k]],
                                          bufs[k], gsems[k]).wait()
                    pltpu.async_copy(bufs[k], acc.at[dst_v.at[j + k]],
                                     ssems[k], add=True)
                for k in range(2):
                    @pl.when(j + 2 + k < half)
                    def _():
                        pltpu.make_async_copy(bufs[k],
                                              acc.at[dst_v.at[j + k]],
                                              ssems[k]).wait()
                        pltpu.async_copy(x_hbm.at[src_v.at[j + 2 + k]],
                                         bufs[k], gsems[k])
                return carry

            lax.fori_loop(0, half // 2, body, 0)
            pltpu.make_async_copy(b0, acc.at[dst_v.at[half - 2]], s0).wait()
            pltpu.make_async_copy(b1, acc.at[dst_v.at[half - 1]], s1).wait()

        plsc.subcore_barrier()
        pltpu.sync_copy(acc.at[pl.ds(s * RP, RP)],
                        out_hbm.at[pl.ds(c * N_PAD + s * RP, RP)])

    return spmv


def _make_deg(nchunk):
    @functools.partial(
        pl.kernel, mesh=_MESH,
        out_type=jax.ShapeDtypeStruct((NC * N_PAD, HD), jnp.float32),
        scratch_types=[
            pltpu.VMEM((nchunk, CB), jnp.int32),
            pltpu.VMEM((CB, HD), jnp.float32),
            pltpu.SemaphoreType.DMA,
            pltpu.VMEM_SHARED((N_PAD, HD), jnp.float32),
        ])
    def deg(dst_hbm, ones_hbm, zeros_hbm, out_hbm, dst_v, ones_v, sem, acc):
        c = lax.axis_index("c")
        s = lax.axis_index("s")
        w = s * NC + c
        pltpu.sync_copy(zeros_hbm, acc.at[pl.ds(s * RP, RP)])
        pltpu.sync_copy(dst_hbm.at[pl.ds(w * nchunk, nchunk)], dst_v)
        pltpu.sync_copy(ones_hbm, ones_v)
        plsc.subcore_barrier()

        def body(j, carry):
            pltpu.async_copy(ones_v, acc.at[dst_v.at[j]], sem, add=True)
            return carry

        lax.fori_loop(0, nchunk, body, 0)

        def drain(j, carry):
            pltpu.make_async_copy(ones_v, acc.at[dst_v.at[0]], sem).wait()
            return carry

        lax.fori_loop(0, nchunk, drain, 0)
        plsc.subcore_barrier()
        pltpu.sync_copy(acc.at[pl.ds(s * RP, RP)],
                        out_hbm.at[pl.ds(c * N_PAD + s * RP, RP)])

    return deg


def _softplus(v):
    return jnp.log1p(jnp.exp(-jnp.abs(v))) + jnp.maximum(v, 0.0)


def _row(cols):
    return pl.BlockSpec((BR, cols), lambda i: (i, 0))


def _full(shape):
    return pl.BlockSpec(shape, lambda i: tuple(0 for _ in shape))


def _part_spec(cols):
    return pl.BlockSpec((NC, BR, cols), lambda i: (0, i, 0))


def _mm(a, b):
    return jnp.dot(a, b, preferred_element_type=jnp.float32)


def _d1_body(x_ref, h0_ref, wpx, bpx, wpr, bpr, wpm, bpm, wps, bps,
             phi_x_ref, pm_ref, ps_ref):
    x = x_ref[...]
    h0 = h0_ref[...]
    phi_x_ref[...] = jnp.maximum(_mm(x, wpx[...]) + bpx[...], 0.0)
    pt = jnp.maximum(_mm(h0, wpr[...]) + bpr[...], 0.0)
    pm_ref[...] = _mm(pt, wpm[...]) + bpm[...]
    ps_ref[...] = _softplus(_mm(pt, wps[...]) + bps[...])


def _d2_body(degp_ref, phi_x_ref, h0_ref, wea, web, dinv_ref, u1_ref):
    deg = degp_ref[0, :, 0:1] + degp_ref[1, :, 0:1] + 1.0
    dinv = 1.0 / jnp.sqrt(jnp.clip(deg, 1.0))
    dinv_ref[...] = jnp.broadcast_to(dinv, dinv_ref.shape)
    u1_ref[...] = dinv * (_mm(phi_x_ref[...], wea[...]) +
                          _mm(h0_ref[...], web[...]))


def _d3_body(s1_ref, u1_ref, dinv_ref, wem, wes, benc, u2_ref):
    dinv = dinv_ref[:, 0:1]
    enc = dinv * (s1_ref[0] + s1_ref[1] + u1_ref[...]) + benc[...]
    u2_ref[...] = dinv * jnp.concatenate(
        [_mm(enc, wem[...]), _mm(enc, wes[...])], axis=1)


def _d4_body(s2_ref, u2_ref, dinv_ref, noise_ref, bem, bes, phi_x_ref,
             h0_ref, wphz, bphz, wxza, wxzb, whz, wxra, wxrb, whr,
             wxha, wxhb, em_ref, es_ref, u3a_ref, u3b_ref, u3c_ref):
    dinv = dinv_ref[:, 0:1]
    t = dinv * (s2_ref[0] + s2_ref[1] + u2_ref[...])
    em = t[:, :ZD] + bem[...]
    es = _softplus(t[:, ZD:] + bes[...])
    em_ref[...] = em
    es_ref[...] = es
    z = em + es * noise_ref[...]
    phi_z = jnp.maximum(_mm(z, wphz[...]) + bphz[...], 0.0)
    phi_x = phi_x_ref[...]
    h0 = h0_ref[...]
    u3a_ref[...] = dinv * (_mm(phi_x, wxza[...]) + _mm(phi_z, wxzb[...]) +
                           _mm(h0, whz[...]))
    u3b_ref[...] = dinv * (_mm(phi_x, wxra[...]) + _mm(phi_z, wxrb[...]) +
                           _mm(h0, whr[...]))
    u3c_ref[...] = dinv * (_mm(phi_x, wxha[...]) + _mm(phi_z, wxhb[...]))


def _d5_body(s3a_ref, s3b_ref, u3a_ref, u3b_ref, dinv_ref, h0_ref, whh,
             zg_ref, u4_ref):
    dinv = dinv_ref[:, 0:1]
    zg = jax.nn.sigmoid(dinv * (s3a_ref[0] + s3a_ref[1] + u3a_ref[...]))
    rg = jax.nn.sigmoid(dinv * (s3b_ref[0] + s3b_ref[1] + u3b_ref[...]))
    zg_ref[...] = zg
    u4_ref[...] = dinv * _mm(rg * h0_ref[...], whh[...])


def _d6_body(s3c_ref, u3c_ref, s4_ref, u4_ref, dinv_ref, zg_ref, h0_ref,
             hout_ref):
    dinv = dinv_ref[:, 0:1]
    ht = jnp.tanh(dinv * (s3c_ref[0] + s3c_ref[1] + u3c_ref[...]) +
                  dinv * (s4_ref[0] + s4_ref[1] + u4_ref[...]))
    zg = zg_ref[...]
    hout_ref[...] = zg * h0_ref[...] + (1.0 - zg) * ht


def _call(body, in_specs, out_specs, out_shapes, args):
    return pl.pallas_call(
        body, grid=(GRID,), in_specs=in_specs, out_specs=out_specs,
        out_shape=out_shapes)(*args)


def kernel(x, edge_index, h, noise, W_phi_x, b_phi_x, W_enc, b_enc,
           W_enc_mean, b_enc_mean, W_enc_std, b_enc_std,
           W_prior, b_prior, W_prior_mean, b_prior_mean,
           W_prior_std, b_prior_std, W_phi_z, b_phi_z,
           W_xz, W_hz, W_xr, W_hr, W_xh, W_hh):
    E = edge_index.shape[1]
    EC = NW * CB
    nchunk = -(-E // EC)
    nchunk += (-nchunk) % 4
    e_pad = nchunk * EC

    f32 = jnp.float32
    padn = N_PAD - N
    xp = jnp.pad(x, ((0, padn), (0, 0)))
    h0p = jnp.pad(h[0], ((0, padn), (0, 0)))
    noisep = jnp.pad(noise, ((0, padn), (0, 0)))

    pad_idx = N + (jnp.arange(e_pad - E, dtype=jnp.int32) % padn)
    srcp = jnp.concatenate([edge_index[0], pad_idx]).reshape(NW * nchunk, CB)
    dstp = jnp.concatenate([edge_index[1], pad_idx]).reshape(NW * nchunk, CB)

    zeros128 = jnp.zeros((RP, HD), f32)
    ones128 = jnp.ones((CB, HD), f32)

    spmv = _make_spmv(nchunk)
    degk = _make_deg(nchunk)

    def b2(v):
        return v.reshape(1, -1)

    degp = degk(dstp, ones128, zeros128).reshape(NC, N_PAD, HD)

    phi_x, pm, ps = _call(
        _d1_body,
        [_row(HD), _row(HD), _full((HD, HD)), _full((1, HD)),
         _full((HD, HD)), _full((1, HD)), _full((HD, ZD)), _full((1, ZD)),
         _full((HD, ZD)), _full((1, ZD))],
        [_row(HD), _row(ZD), _row(ZD)],
        [jax.ShapeDtypeStruct((N_PAD, HD), f32),
         jax.ShapeDtypeStruct((N_PAD, ZD), f32),
         jax.ShapeDtypeStruct((N_PAD, ZD), f32)],
        (xp, h0p, W_phi_x, b2(b_phi_x), W_prior, b2(b_prior),
         W_prior_mean, b2(b_prior_mean), W_prior_std, b2(b_prior_std)))

    dinv16, U1 = _call(
        _d2_body,
        [_part_spec(HD), _row(HD), _row(HD), _full((HD, HD)),
         _full((HD, HD))],
        [_row(16), _row(HD)],
        [jax.ShapeDtypeStruct((N_PAD, 16), f32),
         jax.ShapeDtypeStruct((N_PAD, HD), f32)],
        (degp, phi_x, h0p, W_enc[:HD], W_enc[HD:]))

    S1 = spmv(U1, srcp, dstp, zeros128).reshape(NC, N_PAD, HD)
    U2 = _call(
        _d3_body,
        [_part_spec(HD), _row(HD), _row(16), _full((HD, ZD)),
         _full((HD, ZD)), _full((1, HD))],
        _row(HD),
        jax.ShapeDtypeStruct((N_PAD, HD), f32),
        (S1, U1, dinv16, W_enc_mean, W_enc_std, b2(b_enc)))

    S2 = spmv(U2, srcp, dstp, zeros128).reshape(NC, N_PAD, HD)
    em, es, U3a, U3b, U3c = _call(
        _d4_body,
        [_part_spec(HD), _row(HD), _row(16), _row(ZD), _full((1, ZD)),
         _full((1, ZD)), _row(HD), _row(HD), _full((ZD, HD)),
         _full((1, HD)), _full((HD, HD)), _full((HD, HD)), _full((HD, HD)),
         _full((HD, HD)), _full((HD, HD)), _full((HD, HD)),
         _full((HD, HD)), _full((HD, HD))],
        [_row(ZD), _row(ZD), _row(HD), _row(HD), _row(HD)],
        [jax.ShapeDtypeStruct((N_PAD, ZD), f32),
         jax.ShapeDtypeStruct((N_PAD, ZD), f32),
         jax.ShapeDtypeStruct((N_PAD, HD), f32),
         jax.ShapeDtypeStruct((N_PAD, HD), f32),
         jax.ShapeDtypeStruct((N_PAD, HD), f32)],
        (S2, U2, dinv16, noisep, b2(b_enc_mean), b2(b_enc_std), phi_x, h0p,
         W_phi_z, b2(b_phi_z), W_xz[:HD], W_xz[HD:], W_hz,
         W_xr[:HD], W_xr[HD:], W_hr, W_xh[:HD], W_xh[HD:]))

    S3a = spmv(U3a, srcp, dstp, zeros128).reshape(NC, N_PAD, HD)
    S3b = spmv(U3b, srcp, dstp, zeros128).reshape(NC, N_PAD, HD)
    S3c = spmv(U3c, srcp, dstp, zeros128).reshape(NC, N_PAD, HD)
    zg, U4 = _call(
        _d5_body,
        [_part_spec(HD), _part_spec(HD), _row(HD), _row(HD), _row(16),
         _row(HD), _full((HD, HD))],
        [_row(HD), _row(HD)],
        [jax.ShapeDtypeStruct((N_PAD, HD), f32),
         jax.ShapeDtypeStruct((N_PAD, HD), f32)],
        (S3a, S3b, U3a, U3b, dinv16, h0p, W_hh))

    S4 = spmv(U4, srcp, dstp, zeros128).reshape(NC, N_PAD, HD)
    hout = _call(
        _d6_body,
        [_part_spec(HD), _row(HD), _part_spec(HD), _row(HD), _row(16),
         _row(HD), _row(HD)],
        _row(HD),
        jax.ShapeDtypeStruct((N_PAD, HD), f32),
        (S3c, U3c, S4, U4, dinv16, zg, h0p))

    return (em[:N], es[:N], pm[:N], ps[:N], hout[:N][None])


# speedup vs baseline: 21.4227x; 1.0152x over previous
import functools

import jax
import jax.numpy as jnp
from jax import lax
from jax.experimental import pallas as pl
from jax.experimental.pallas import tpu as pltpu
from jax.experimental.pallas import tpu_sc as plsc

N = 10000
N_PAD = 10240
HD = 128
ZD = 64
NC, NS = 2, 16
NW = NC * NS
RP = N_PAD // NS
BR = 1024
GRID = N_PAD // BR

_MESH = plsc.VectorSubcoreMesh(core_axis_name="c", subcore_axis_name="s")


def _chunking(E):
    epw = -(-E // NW)
    epw += (-epw) % 4
    for cb in range(min(128, epw), 63, -1):
        if epw % cb == 0 and (epw // cb) % 4 == 0:
            return cb, epw // cb, epw
    epw += (-epw) % 512
    return 128, epw // 128, epw


def _make_spmv(nchunk, CB):
    assert nchunk % 4 == 0
    half = nchunk // 2

    @functools.partial(
        pl.kernel, mesh=_MESH,
        out_type=jax.ShapeDtypeStruct((NC * N_PAD, HD), jnp.float32),
        scratch_types=[
            pltpu.VMEM((half, CB), jnp.int32),
            pltpu.VMEM((half, CB), jnp.int32),
            pltpu.VMEM((CB, HD), jnp.float32),
            pltpu.VMEM((CB, HD), jnp.float32),
            pltpu.SemaphoreType.DMA,
            pltpu.SemaphoreType.DMA,
            pltpu.SemaphoreType.DMA,
            pltpu.SemaphoreType.DMA,
            pltpu.VMEM_SHARED((N_PAD, HD), jnp.float32),
        ])
    def spmv(x_hbm, src_hbm, dst_hbm, zeros_hbm, out_hbm,
             src_v, dst_v, b0, b1, g0, g1, s0, s1, acc):
        c = lax.axis_index("c")
        s = lax.axis_index("s")
        w = s * NC + c
        pltpu.sync_copy(zeros_hbm, acc.at[pl.ds(s * RP, RP)])
        plsc.subcore_barrier()
        bufs = (b0, b1)
        gsems = (g0, g1)
        ssems = (s0, s1)

        for phase in range(2):
            base = w * nchunk + phase * half
            pltpu.sync_copy(src_hbm.at[pl.ds(base, half)], src_v)
            pltpu.sync_copy(dst_hbm.at[pl.ds(base, half)], dst_v)
            pltpu.async_copy(x_hbm.at[src_v.at[0]], b0, g0)
            pltpu.async_copy(x_hbm.at[src_v.at[1]], b1, g1)

            def body(j2, carry):
                j = j2 * 2
                for k in range(2):
                    pltpu.make_async_copy(x_hbm.at[src_v.at[j + k]],
                                          bufs[k], gsems[k]).wait()
                    pltpu.async_copy(bufs[k], acc.at[dst_v.at[j + k]],
                                     ssems[k], add=True)
                for k in range(2):
                    @pl.when(j + 2 + k < half)
                    def _():
                        pltpu.make_async_copy(bufs[k],
                                              acc.at[dst_v.at[j + k]],
                                              ssems[k]).wait()
                        pltpu.async_copy(x_hbm.at[src_v.at[j + 2 + k]],
                                         bufs[k], gsems[k])
                return carry

            lax.fori_loop(0, half // 2, body, 0)
            pltpu.make_async_copy(b0, acc.at[dst_v.at[half - 2]], s0).wait()
            pltpu.make_async_copy(b1, acc.at[dst_v.at[half - 1]], s1).wait()

        plsc.subcore_barrier()
        pltpu.sync_copy(acc.at[pl.ds(s * RP, RP)],
                        out_hbm.at[pl.ds(c * N_PAD + s * RP, RP)])

    return spmv


def _make_deg(nchunk, CB):
    @functools.partial(
        pl.kernel, mesh=_MESH,
        out_type=jax.ShapeDtypeStruct((NC * N_PAD, HD), jnp.float32),
        scratch_types=[
            pltpu.VMEM((nchunk, CB), jnp.int32),
            pltpu.VMEM((CB, HD), jnp.float32),
            pltpu.SemaphoreType.DMA,
            pltpu.VMEM_SHARED((N_PAD, HD), jnp.float32),
        ])
    def deg(dst_hbm, ones_hbm, zeros_hbm, out_hbm, dst_v, ones_v, sem, acc):
        c = lax.axis_index("c")
        s = lax.axis_index("s")
        w = s * NC + c
        pltpu.sync_copy(zeros_hbm, acc.at[pl.ds(s * RP, RP)])
        pltpu.sync_copy(dst_hbm.at[pl.ds(w * nchunk, nchunk)], dst_v)
        pltpu.sync_copy(ones_hbm, ones_v)
        plsc.subcore_barrier()

        def body(j, carry):
            pltpu.async_copy(ones_v, acc.at[dst_v.at[j]], sem, add=True)
            return carry

        lax.fori_loop(0, nchunk, body, 0)

        def drain(j, carry):
            pltpu.make_async_copy(ones_v, acc.at[dst_v.at[0]], sem).wait()
            return carry

        lax.fori_loop(0, nchunk, drain, 0)
        plsc.subcore_barrier()
        pltpu.sync_copy(acc.at[pl.ds(s * RP, RP)],
                        out_hbm.at[pl.ds(c * N_PAD + s * RP, RP)])

    return deg


def _softplus(v):
    return jnp.log1p(jnp.exp(-jnp.abs(v))) + jnp.maximum(v, 0.0)


def _row(cols):
    return pl.BlockSpec((BR, cols), lambda i: (i, 0))


def _full(shape):
    return pl.BlockSpec(shape, lambda i: tuple(0 for _ in shape))


def _part_spec(cols):
    return pl.BlockSpec((NC, BR, cols), lambda i: (0, i, 0))


def _mm(a, b):
    return jnp.dot(a, b, preferred_element_type=jnp.float32)


def _d1_body(x_ref, h0_ref, wpx, bpx, wpr, bpr, wpm, bpm, wps, bps,
             phi_x_ref, pm_ref, ps_ref):
    x = x_ref[...]
    h0 = h0_ref[...]
    phi_x_ref[...] = jnp.maximum(_mm(x, wpx[...]) + bpx[...], 0.0)
    pt = jnp.maximum(_mm(h0, wpr[...]) + bpr[...], 0.0)
    pm_ref[...] = _mm(pt, wpm[...]) + bpm[...]
    ps_ref[...] = _softplus(_mm(pt, wps[...]) + bps[...])


def _d2_body(degp_ref, phi_x_ref, h0_ref, wea, web, dinv_ref, u1_ref):
    deg = degp_ref[0, :, 0:1] + degp_ref[1, :, 0:1] + 1.0
    dinv = 1.0 / jnp.sqrt(jnp.clip(deg, 1.0))
    dinv_ref[...] = jnp.broadcast_to(dinv, dinv_ref.shape)
    u1_ref[...] = dinv * (_mm(phi_x_ref[...], wea[...]) +
                          _mm(h0_ref[...], web[...]))


def _d3_body(s1_ref, u1_ref, dinv_ref, wem, wes, benc, u2_ref):
    dinv = dinv_ref[:, 0:1]
    enc = dinv * (s1_ref[0] + s1_ref[1] + u1_ref[...]) + benc[...]
    u2_ref[...] = dinv * jnp.concatenate(
        [_mm(enc, wem[...]), _mm(enc, wes[...])], axis=1)


def _d4_body(s2_ref, u2_ref, dinv_ref, noise_ref, bem, bes, phi_x_ref,
             h0_ref, wphz, bphz, wxza, wxzb, whz, wxra, wxrb, whr,
             wxha, wxhb, em_ref, es_ref, u3a_ref, u3b_ref, u3c_ref):
    dinv = dinv_ref[:, 0:1]
    t = dinv * (s2_ref[0] + s2_ref[1] + u2_ref[...])
    em = t[:, :ZD] + bem[...]
    es = _softplus(t[:, ZD:] + bes[...])
    em_ref[...] = em
    es_ref[...] = es
    z = em + es * noise_ref[...]
    phi_z = jnp.maximum(_mm(z, wphz[...]) + bphz[...], 0.0)
    phi_x = phi_x_ref[...]
    h0 = h0_ref[...]
    u3a_ref[...] = dinv * (_mm(phi_x, wxza[...]) + _mm(phi_z, wxzb[...]) +
                           _mm(h0, whz[...]))
    u3b_ref[...] = dinv * (_mm(phi_x, wxra[...]) + _mm(phi_z, wxrb[...]) +
                           _mm(h0, whr[...]))
    u3c_ref[...] = dinv * (_mm(phi_x, wxha[...]) + _mm(phi_z, wxhb[...]))


def _d5_body(s3a_ref, s3b_ref, u3a_ref, u3b_ref, dinv_ref, h0_ref, whh,
             zg_ref, u4_ref):
    dinv = dinv_ref[:, 0:1]
    zg = jax.nn.sigmoid(dinv * (s3a_ref[0] + s3a_ref[1] + u3a_ref[...]))
    rg = jax.nn.sigmoid(dinv * (s3b_ref[0] + s3b_ref[1] + u3b_ref[...]))
    zg_ref[...] = zg
    u4_ref[...] = dinv * _mm(rg * h0_ref[...], whh[...])


def _d6_body(s3c_ref, u3c_ref, s4_ref, u4_ref, dinv_ref, zg_ref, h0_ref,
             hout_ref):
    dinv = dinv_ref[:, 0:1]
    ht = jnp.tanh(dinv * (s3c_ref[0] + s3c_ref[1] + u3c_ref[...]) +
                  dinv * (s4_ref[0] + s4_ref[1] + u4_ref[...]))
    zg = zg_ref[...]
    hout_ref[...] = zg * h0_ref[...] + (1.0 - zg) * ht


def _call(body, in_specs, out_specs, out_shapes, args):
    return pl.pallas_call(
        body, grid=(GRID,), in_specs=in_specs, out_specs=out_specs,
        out_shape=out_shapes)(*args)


def kernel(x, edge_index, h, noise, W_phi_x, b_phi_x, W_enc, b_enc,
           W_enc_mean, b_enc_mean, W_enc_std, b_enc_std,
           W_prior, b_prior, W_prior_mean, b_prior_mean,
           W_prior_std, b_prior_std, W_phi_z, b_phi_z,
           W_xz, W_hz, W_xr, W_hr, W_xh, W_hh):
    E = edge_index.shape[1]
    CB, nchunk, epw = _chunking(E)
    e_pad = NW * epw

    f32 = jnp.float32
    padn = N_PAD - N
    xp = jnp.pad(x, ((0, padn), (0, 0)))
    h0p = jnp.pad(h[0], ((0, padn), (0, 0)))
    noisep = jnp.pad(noise, ((0, padn), (0, 0)))

    pad_idx = N + (jnp.arange(e_pad - E, dtype=jnp.int32) % padn)
    srcp = jnp.concatenate([edge_index[0], pad_idx]).reshape(NW * nchunk, CB)
    dstp = jnp.concatenate([edge_index[1], pad_idx]).reshape(NW * nchunk, CB)

    zeros128 = jnp.zeros((RP, HD), f32)
    ones128 = jnp.ones((CB, HD), f32)

    spmv = _make_spmv(nchunk, CB)
    degk = _make_deg(nchunk, CB)

    def b2(v):
        return v.reshape(1, -1)

    degp = degk(dstp, ones128, zeros128).reshape(NC, N_PAD, HD)

    phi_x, pm, ps = _call(
        _d1_body,
        [_row(HD), _row(HD), _full((HD, HD)), _full((1, HD)),
         _full((HD, HD)), _full((1, HD)), _full((HD, ZD)), _full((1, ZD)),
         _full((HD, ZD)), _full((1, ZD))],
        [_row(HD), _row(ZD), _row(ZD)],
        [jax.ShapeDtypeStruct((N_PAD, HD), f32),
         jax.ShapeDtypeStruct((N_PAD, ZD), f32),
         jax.ShapeDtypeStruct((N_PAD, ZD), f32)],
        (xp, h0p, W_phi_x, b2(b_phi_x), W_prior, b2(b_prior),
         W_prior_mean, b2(b_prior_mean), W_prior_std, b2(b_prior_std)))

    dinv16, U1 = _call(
        _d2_body,
        [_part_spec(HD), _row(HD), _row(HD), _full((HD, HD)),
         _full((HD, HD))],
        [_row(16), _row(HD)],
        [jax.ShapeDtypeStruct((N_PAD, 16), f32),
         jax.ShapeDtypeStruct((N_PAD, HD), f32)],
        (degp, phi_x, h0p, W_enc[:HD], W_enc[HD:]))

    S1 = spmv(U1, srcp, dstp, zeros128).reshape(NC, N_PAD, HD)
    U2 = _call(
        _d3_body,
        [_part_spec(HD), _row(HD), _row(16), _full((HD, ZD)),
         _full((HD, ZD)), _full((1, HD))],
        _row(HD),
        jax.ShapeDtypeStruct((N_PAD, HD), f32),
        (S1, U1, dinv16, W_enc_mean, W_enc_std, b2(b_enc)))

    S2 = spmv(U2, srcp, dstp, zeros128).reshape(NC, N_PAD, HD)
    em, es, U3a, U3b, U3c = _call(
        _d4_body,
        [_part_spec(HD), _row(HD), _row(16), _row(ZD), _full((1, ZD)),
         _full((1, ZD)), _row(HD), _row(HD), _full((ZD, HD)),
         _full((1, HD)), _full((HD, HD)), _full((HD, HD)), _full((HD, HD)),
         _full((HD, HD)), _full((HD, HD)), _full((HD, HD)),
         _full((HD, HD)), _full((HD, HD))],
        [_row(ZD), _row(ZD), _row(HD), _row(HD), _row(HD)],
        [jax.ShapeDtypeStruct((N_PAD, ZD), f32),
         jax.ShapeDtypeStruct((N_PAD, ZD), f32),
         jax.ShapeDtypeStruct((N_PAD, HD), f32),
         jax.ShapeDtypeStruct((N_PAD, HD), f32),
         jax.ShapeDtypeStruct((N_PAD, HD), f32)],
        (S2, U2, dinv16, noisep, b2(b_enc_mean), b2(b_enc_std), phi_x, h0p,
         W_phi_z, b2(b_phi_z), W_xz[:HD], W_xz[HD:], W_hz,
         W_xr[:HD], W_xr[HD:], W_hr, W_xh[:HD], W_xh[HD:]))

    S3a = spmv(U3a, srcp, dstp, zeros128).reshape(NC, N_PAD, HD)
    S3b = spmv(U3b, srcp, dstp, zeros128).reshape(NC, N_PAD, HD)
    S3c = spmv(U3c, srcp, dstp, zeros128).reshape(NC, N_PAD, HD)
    zg, U4 = _call(
        _d5_body,
        [_part_spec(HD), _part_spec(HD), _row(HD), _row(HD), _row(16),
         _row(HD), _full((HD, HD))],
        [_row(HD), _row(HD)],
        [jax.ShapeDtypeStruct((N_PAD, HD), f32),
         jax.ShapeDtypeStruct((N_PAD, HD), f32)],
        (S3a, S3b, U3a, U3b, dinv16, h0p, W_hh))

    S4 = spmv(U4, srcp, dstp, zeros128).reshape(NC, N_PAD, HD)
    hout = _call(
        _d6_body,
        [_part_spec(HD), _row(HD), _part_spec(HD), _row(HD), _row(16),
         _row(HD), _row(HD)],
        _row(HD),
        jax.ShapeDtypeStruct((N_PAD, HD), f32),
        (S3c, U3c, S4, U4, dinv16, zg, h0p))

    return (em[:N], es[:N], pm[:N], ps[:N], hout[:N][None])


# speedup vs baseline: 21.5353x; 1.0053x over previous
import functools

import jax
import jax.numpy as jnp
from jax import lax
from jax.experimental import pallas as pl
from jax.experimental.pallas import tpu as pltpu
from jax.experimental.pallas import tpu_sc as plsc

N = 10000
N_PAD = 10240
HD = 128
ZD = 64
NC, NS = 2, 16
NW = NC * NS
RP = N_PAD // NS
BR = 1024
GRID = N_PAD // BR

_MESH = plsc.VectorSubcoreMesh(core_axis_name="c", subcore_axis_name="s")


def _chunking(E):
    epw = -(-E // NW)
    epw += (-epw) % 4
    for cb in range(min(128, epw), 63, -1):
        if epw % cb == 0 and (epw // cb) % 4 == 0:
            return cb, epw // cb, epw
    epw += (-epw) % 512
    return 128, epw // 128, epw


def _make_spmv(nchunk, CB):
    assert nchunk % 4 == 0
    half = nchunk // 2

    @functools.partial(
        pl.kernel, mesh=_MESH,
        out_type=jax.ShapeDtypeStruct((NC * N_PAD, HD), jnp.float32),
        scratch_types=[
            pltpu.VMEM((half, CB), jnp.int32),
            pltpu.VMEM((half, CB), jnp.int32),
            pltpu.VMEM((CB, HD), jnp.float32),
            pltpu.VMEM((CB, HD), jnp.float32),
            pltpu.SemaphoreType.DMA,
            pltpu.SemaphoreType.DMA,
            pltpu.SemaphoreType.DMA,
            pltpu.SemaphoreType.DMA,
            pltpu.VMEM_SHARED((N_PAD, HD), jnp.float32),
        ])
    def spmv(x_hbm, src_hbm, dst_hbm, zeros_hbm, out_hbm,
             src_v, dst_v, b0, b1, g0, g1, s0, s1, acc):
        c = lax.axis_index("c")
        s = lax.axis_index("s")
        w = s * NC + c
        pltpu.async_copy(zeros_hbm, acc.at[pl.ds(s * RP, RP)], s0)
        pltpu.async_copy(src_hbm.at[pl.ds(w * nchunk, half)], src_v, g0)
        pltpu.async_copy(dst_hbm.at[pl.ds(w * nchunk, half)], dst_v, g1)
        pltpu.make_async_copy(zeros_hbm, acc.at[pl.ds(s * RP, RP)], s0).wait()
        pltpu.make_async_copy(src_hbm.at[pl.ds(w * nchunk, half)], src_v,
                              g0).wait()
        pltpu.make_async_copy(dst_hbm.at[pl.ds(w * nchunk, half)], dst_v,
                              g1).wait()
        plsc.subcore_barrier()
        bufs = (b0, b1)
        gsems = (g0, g1)
        ssems = (s0, s1)

        for phase in range(2):
            base = w * nchunk + phase * half
            if phase:
                pltpu.sync_copy(src_hbm.at[pl.ds(base, half)], src_v)
                pltpu.sync_copy(dst_hbm.at[pl.ds(base, half)], dst_v)
            pltpu.async_copy(x_hbm.at[src_v.at[0]], b0, g0)
            pltpu.async_copy(x_hbm.at[src_v.at[1]], b1, g1)

            def body(j2, carry):
                j = j2 * 2
                for k in range(2):
                    pltpu.make_async_copy(x_hbm.at[src_v.at[j + k]],
                                          bufs[k], gsems[k]).wait()
                    pltpu.async_copy(bufs[k], acc.at[dst_v.at[j + k]],
                                     ssems[k], add=True)
                for k in range(2):
                    @pl.when(j + 2 + k < half)
                    def _():
                        pltpu.make_async_copy(bufs[k],
                                              acc.at[dst_v.at[j + k]],
                                              ssems[k]).wait()
                        pltpu.async_copy(x_hbm.at[src_v.at[j + 2 + k]],
                                         bufs[k], gsems[k])
                return carry

            lax.fori_loop(0, half // 2, body, 0)
            pltpu.make_async_copy(b0, acc.at[dst_v.at[half - 2]], s0).wait()
            pltpu.make_async_copy(b1, acc.at[dst_v.at[half - 1]], s1).wait()

        plsc.subcore_barrier()
        pltpu.sync_copy(acc.at[pl.ds(s * RP, RP)],
                        out_hbm.at[pl.ds(c * N_PAD + s * RP, RP)])

    return spmv


def _make_deg(nchunk, CB):
    @functools.partial(
        pl.kernel, mesh=_MESH,
        out_type=jax.ShapeDtypeStruct((NC * N_PAD, HD), jnp.float32),
        scratch_types=[
            pltpu.VMEM((nchunk, CB), jnp.int32),
            pltpu.VMEM((CB, HD), jnp.float32),
            pltpu.SemaphoreType.DMA,
            pltpu.VMEM_SHARED((N_PAD, HD), jnp.float32),
        ])
    def deg(dst_hbm, ones_hbm, zeros_hbm, out_hbm, dst_v, ones_v, sem, acc):
        c = lax.axis_index("c")
        s = lax.axis_index("s")
        w = s * NC + c
        pltpu.async_copy(zeros_hbm, acc.at[pl.ds(s * RP, RP)], sem)
        pltpu.async_copy(dst_hbm.at[pl.ds(w * nchunk, nchunk)], dst_v, sem)
        pltpu.async_copy(ones_hbm, ones_v, sem)
        pltpu.make_async_copy(zeros_hbm, acc.at[pl.ds(s * RP, RP)], sem).wait()
        pltpu.make_async_copy(dst_hbm.at[pl.ds(w * nchunk, nchunk)], dst_v,
                              sem).wait()
        pltpu.make_async_copy(ones_hbm, ones_v, sem).wait()
        plsc.subcore_barrier()

        def body(j, carry):
            pltpu.async_copy(ones_v, acc.at[dst_v.at[j]], sem, add=True)
            return carry

        lax.fori_loop(0, nchunk, body, 0)

        def drain(j, carry):
            pltpu.make_async_copy(ones_v, acc.at[dst_v.at[0]], sem).wait()
            return carry

        lax.fori_loop(0, nchunk, drain, 0)
        plsc.subcore_barrier()
        pltpu.sync_copy(acc.at[pl.ds(s * RP, RP)],
                        out_hbm.at[pl.ds(c * N_PAD + s * RP, RP)])

    return deg


def _softplus(v):
    return jnp.log1p(jnp.exp(-jnp.abs(v))) + jnp.maximum(v, 0.0)


def _row(cols):
    return pl.BlockSpec((BR, cols), lambda i: (i, 0))


def _full(shape):
    return pl.BlockSpec(shape, lambda i: tuple(0 for _ in shape))


def _part_spec(cols):
    return pl.BlockSpec((NC, BR, cols), lambda i: (0, i, 0))


def _mm(a, b):
    return jnp.dot(a, b, preferred_element_type=jnp.float32)


def _d1_body(x_ref, h0_ref, wpx, bpx, wpr, bpr, wpm, bpm, wps, bps,
             phi_x_ref, pm_ref, ps_ref):
    x = x_ref[...]
    h0 = h0_ref[...]
    phi_x_ref[...] = jnp.maximum(_mm(x, wpx[...]) + bpx[...], 0.0)
    pt = jnp.maximum(_mm(h0, wpr[...]) + bpr[...], 0.0)
    pm_ref[...] = _mm(pt, wpm[...]) + bpm[...]
    ps_ref[...] = _softplus(_mm(pt, wps[...]) + bps[...])


def _d2_body(degp_ref, phi_x_ref, h0_ref, wea, web, dinv_ref, u1_ref):
    deg = degp_ref[0, :, 0:1] + degp_ref[1, :, 0:1] + 1.0
    dinv = 1.0 / jnp.sqrt(jnp.clip(deg, 1.0))
    dinv_ref[...] = jnp.broadcast_to(dinv, dinv_ref.shape)
    u1_ref[...] = dinv * (_mm(phi_x_ref[...], wea[...]) +
                          _mm(h0_ref[...], web[...]))


def _d3_body(s1_ref, u1_ref, dinv_ref, wem, wes, benc, u2_ref):
    dinv = dinv_ref[:, 0:1]
    enc = dinv * (s1_ref[0] + s1_ref[1] + u1_ref[...]) + benc[...]
    u2_ref[...] = dinv * jnp.concatenate(
        [_mm(enc, wem[...]), _mm(enc, wes[...])], axis=1)


def _d4_body(s2_ref, u2_ref, dinv_ref, noise_ref, bem, bes, phi_x_ref,
             h0_ref, wphz, bphz, wxza, wxzb, whz, wxra, wxrb, whr,
             wxha, wxhb, em_ref, es_ref, u3a_ref, u3b_ref, u3c_ref):
    dinv = dinv_ref[:, 0:1]
    t = dinv * (s2_ref[0] + s2_ref[1] + u2_ref[...])
    em = t[:, :ZD] + bem[...]
    es = _softplus(t[:, ZD:] + bes[...])
    em_ref[...] = em
    es_ref[...] = es
    z = em + es * noise_ref[...]
    phi_z = jnp.maximum(_mm(z, wphz[...]) + bphz[...], 0.0)
    phi_x = phi_x_ref[...]
    h0 = h0_ref[...]
    u3a_ref[...] = dinv * (_mm(phi_x, wxza[...]) + _mm(phi_z, wxzb[...]) +
                           _mm(h0, whz[...]))
    u3b_ref[...] = dinv * (_mm(phi_x, wxra[...]) + _mm(phi_z, wxrb[...]) +
                           _mm(h0, whr[...]))
    u3c_ref[...] = dinv * (_mm(phi_x, wxha[...]) + _mm(phi_z, wxhb[...]))


def _d5_body(s3a_ref, s3b_ref, u3a_ref, u3b_ref, dinv_ref, h0_ref, whh,
             zg_ref, u4_ref):
    dinv = dinv_ref[:, 0:1]
    zg = jax.nn.sigmoid(dinv * (s3a_ref[0] + s3a_ref[1] + u3a_ref[...]))
    rg = jax.nn.sigmoid(dinv * (s3b_ref[0] + s3b_ref[1] + u3b_ref[...]))
    zg_ref[...] = zg
    u4_ref[...] = dinv * _mm(rg * h0_ref[...], whh[...])


def _d6_body(s3c_ref, u3c_ref, s4_ref, u4_ref, dinv_ref, zg_ref, h0_ref,
             hout_ref):
    dinv = dinv_ref[:, 0:1]
    ht = jnp.tanh(dinv * (s3c_ref[0] + s3c_ref[1] + u3c_ref[...]) +
                  dinv * (s4_ref[0] + s4_ref[1] + u4_ref[...]))
    zg = zg_ref[...]
    hout_ref[...] = zg * h0_ref[...] + (1.0 - zg) * ht


def _call(body, in_specs, out_specs, out_shapes, args):
    return pl.pallas_call(
        body, grid=(GRID,), in_specs=in_specs, out_specs=out_specs,
        out_shape=out_shapes)(*args)


def kernel(x, edge_index, h, noise, W_phi_x, b_phi_x, W_enc, b_enc,
           W_enc_mean, b_enc_mean, W_enc_std, b_enc_std,
           W_prior, b_prior, W_prior_mean, b_prior_mean,
           W_prior_std, b_prior_std, W_phi_z, b_phi_z,
           W_xz, W_hz, W_xr, W_hr, W_xh, W_hh):
    E = edge_index.shape[1]
    CB, nchunk, epw = _chunking(E)
    e_pad = NW * epw

    f32 = jnp.float32
    padn = N_PAD - N
    xp = jnp.pad(x, ((0, padn), (0, 0)))
    h0p = jnp.pad(h[0], ((0, padn), (0, 0)))
    noisep = jnp.pad(noise, ((0, padn), (0, 0)))

    pad_idx = N + (jnp.arange(e_pad - E, dtype=jnp.int32) % padn)
    srcp = jnp.concatenate([edge_index[0], pad_idx]).reshape(NW * nchunk, CB)
    dstp = jnp.concatenate([edge_index[1], pad_idx]).reshape(NW * nchunk, CB)

    zeros128 = jnp.zeros((RP, HD), f32)
    ones128 = jnp.ones((CB, HD), f32)

    spmv = _make_spmv(nchunk, CB)
    degk = _make_deg(nchunk, CB)

    def b2(v):
        return v.reshape(1, -1)

    degp = degk(dstp, ones128, zeros128).reshape(NC, N_PAD, HD)

    phi_x, pm, ps = _call(
        _d1_body,
        [_row(HD), _row(HD), _full((HD, HD)), _full((1, HD)),
         _full((HD, HD)), _full((1, HD)), _full((HD, ZD)), _full((1, ZD)),
         _full((HD, ZD)), _full((1, ZD))],
        [_row(HD), _row(ZD), _row(ZD)],
        [jax.ShapeDtypeStruct((N_PAD, HD), f32),
         jax.ShapeDtypeStruct((N_PAD, ZD), f32),
         jax.ShapeDtypeStruct((N_PAD, ZD), f32)],
        (xp, h0p, W_phi_x, b2(b_phi_x), W_prior, b2(b_prior),
         W_prior_mean, b2(b_prior_mean), W_prior_std, b2(b_prior_std)))

    dinv16, U1 = _call(
        _d2_body,
        [_part_spec(HD), _row(HD), _row(HD), _full((HD, HD)),
         _full((HD, HD))],
        [_row(16), _row(HD)],
        [jax.ShapeDtypeStruct((N_PAD, 16), f32),
         jax.ShapeDtypeStruct((N_PAD, HD), f32)],
        (degp, phi_x, h0p, W_enc[:HD], W_enc[HD:]))

    S1 = spmv(U1, srcp, dstp, zeros128).reshape(NC, N_PAD, HD)
    U2 = _call(
        _d3_body,
        [_part_spec(HD), _row(HD), _row(16), _full((HD, ZD)),
         _full((HD, ZD)), _full((1, HD))],
        _row(HD),
        jax.ShapeDtypeStruct((N_PAD, HD), f32),
        (S1, U1, dinv16, W_enc_mean, W_enc_std, b2(b_enc)))

    S2 = spmv(U2, srcp, dstp, zeros128).reshape(NC, N_PAD, HD)
    em, es, U3a, U3b, U3c = _call(
        _d4_body,
        [_part_spec(HD), _row(HD), _row(16), _row(ZD), _full((1, ZD)),
         _full((1, ZD)), _row(HD), _row(HD), _full((ZD, HD)),
         _full((1, HD)), _full((HD, HD)), _full((HD, HD)), _full((HD, HD)),
         _full((HD, HD)), _full((HD, HD)), _full((HD, HD)),
         _full((HD, HD)), _full((HD, HD))],
        [_row(ZD), _row(ZD), _row(HD), _row(HD), _row(HD)],
        [jax.ShapeDtypeStruct((N_PAD, ZD), f32),
         jax.ShapeDtypeStruct((N_PAD, ZD), f32),
         jax.ShapeDtypeStruct((N_PAD, HD), f32),
         jax.ShapeDtypeStruct((N_PAD, HD), f32),
         jax.ShapeDtypeStruct((N_PAD, HD), f32)],
        (S2, U2, dinv16, noisep, b2(b_enc_mean), b2(b_enc_std), phi_x, h0p,
         W_phi_z, b2(b_phi_z), W_xz[:HD], W_xz[HD:], W_hz,
         W_xr[:HD], W_xr[HD:], W_hr, W_xh[:HD], W_xh[HD:]))

    S3a = spmv(U3a, srcp, dstp, zeros128).reshape(NC, N_PAD, HD)
    S3b = spmv(U3b, srcp, dstp, zeros128).reshape(NC, N_PAD, HD)
    S3c = spmv(U3c, srcp, dstp, zeros128).reshape(NC, N_PAD, HD)
    zg, U4 = _call(
        _d5_body,
        [_part_spec(HD), _part_spec(HD), _row(HD), _row(HD), _row(16),
         _row(HD), _full((HD, HD))],
        [_row(HD), _row(HD)],
        [jax.ShapeDtypeStruct((N_PAD, HD), f32),
         jax.ShapeDtypeStruct((N_PAD, HD), f32)],
        (S3a, S3b, U3a, U3b, dinv16, h0p, W_hh))

    S4 = spmv(U4, srcp, dstp, zeros128).reshape(NC, N_PAD, HD)
    hout = _call(
        _d6_body,
        [_part_spec(HD), _row(HD), _part_spec(HD), _row(HD), _row(16),
         _row(HD), _row(HD)],
        _row(HD),
        jax.ShapeDtypeStruct((N_PAD, HD), f32),
        (S3c, U3c, S4, U4, dinv16, zg, h0p))

    return (em[:N], es[:N], pm[:N], ps[:N], hout[:N][None])
